# initial kernel scaffold (unmeasured)
import jax
import jax.numpy as jnp
from jax import lax
from jax.experimental import pallas as pl
from jax.experimental.pallas import tpu as pltpu

N_DEV = 32
_GELU_C = 0.7978845608028654


def kernel(x, w_mat):
    m_per, k = x.shape
    _, n = w_mat.shape
    n_per = n // N_DEV

    def body(x_ref, w_ref, out_ref, y_ref, comm_ref, send_sems, recv_sems):
        my = lax.axis_index("i")

        xb = x_ref[...].astype(jnp.bfloat16)
        wb = w_ref[...].astype(jnp.bfloat16)
        y = jnp.dot(xb, wb, preferred_element_type=jnp.float32)
        y = 0.5 * y * (1.0 + jnp.tanh(_GELU_C * (y + 0.044715 * (y * y * y))))
        y_ref[...] = y.astype(jnp.bfloat16)

        rdmas = []
        for d in range(1, N_DEV):
            tgt = (my + d) % N_DEV
            rdma = pltpu.make_async_remote_copy(
                src_ref=y_ref.at[:, pl.ds(tgt * n_per, n_per)],
                dst_ref=comm_ref.at[d],
                send_sem=send_sems.at[d],
                recv_sem=recv_sems.at[d],
                device_id=(tgt,),
                device_id_type=pl.DeviceIdType.MESH,
            )
            rdma.start()
            rdmas.append(rdma)

        out_ref[pl.ds(my * m_per, m_per), :] = y_ref[
            :, pl.ds(my * n_per, n_per)
        ].astype(jnp.float32)

        for d in range(1, N_DEV):
            rdmas[d - 1].wait_recv()
            origin = (my - d) % N_DEV
            out_ref[pl.ds(origin * m_per, m_per), :] = comm_ref[d].astype(
                jnp.float32
            )
        for d in range(1, N_DEV):
            rdmas[d - 1].wait_send()

    return pl.pallas_call(
        body,
        out_shape=jax.ShapeDtypeStruct((N_DEV * m_per, n_per), jnp.float32),
        in_specs=[
            pl.BlockSpec(memory_space=pltpu.VMEM),
            pl.BlockSpec(memory_space=pltpu.VMEM),
        ],
        out_specs=pl.BlockSpec(memory_space=pltpu.VMEM),
        scratch_shapes=[
            pltpu.VMEM((m_per, n), jnp.bfloat16),
            pltpu.VMEM((N_DEV, m_per, n_per), jnp.bfloat16),
            pltpu.SemaphoreType.DMA((N_DEV,)),
            pltpu.SemaphoreType.DMA((N_DEV,)),
        ],
    )(x, w_mat)


# baseline (device time: 43961 ns/iter reference)
import jax
import jax.numpy as jnp
from jax import lax
from jax.experimental import pallas as pl
from jax.experimental.pallas import tpu as pltpu

N_DEV = 32
_GELU_C = 0.7978845608028654


def kernel(x, w_mat):
    m_per, k = x.shape
    _, n = w_mat.shape
    n_per = n // N_DEV

    def body(x_ref, w_ref, out_ref, y_ref, comm_ref, send_sems, recv_sems):
        my = lax.axis_index("i")

        xb = x_ref[...].astype(jnp.bfloat16)
        wb = w_ref[...].astype(jnp.bfloat16)
        y = jnp.dot(xb, wb, preferred_element_type=jnp.float32)
        y = 0.5 * y * (1.0 + jnp.tanh(_GELU_C * (y + 0.044715 * (y * y * y))))
        yb = y.astype(jnp.bfloat16)
        for j in range(N_DEV):
            y_ref[j] = yb[:, j * n_per : (j + 1) * n_per]

        rdmas = []
        for d in range(1, N_DEV):
            tgt = (my + d) % N_DEV
            rdma = pltpu.make_async_remote_copy(
                src_ref=y_ref.at[tgt],
                dst_ref=comm_ref.at[d],
                send_sem=send_sems.at[d],
                recv_sem=recv_sems.at[d],
                device_id=(tgt,),
                device_id_type=pl.DeviceIdType.MESH,
            )
            rdma.start()
            rdmas.append(rdma)

        out_ref[pl.ds(my * m_per, m_per), :] = y_ref[my].astype(jnp.float32)

        for d in range(1, N_DEV):
            rdmas[d - 1].wait_recv()
            origin = (my - d) % N_DEV
            out_ref[pl.ds(origin * m_per, m_per), :] = comm_ref[d].astype(
                jnp.float32
            )
        for d in range(1, N_DEV):
            rdmas[d - 1].wait_send()

    return pl.pallas_call(
        body,
        out_shape=jax.ShapeDtypeStruct((N_DEV * m_per, n_per), jnp.float32),
        in_specs=[
            pl.BlockSpec(memory_space=pltpu.VMEM),
            pl.BlockSpec(memory_space=pltpu.VMEM),
        ],
        out_specs=pl.BlockSpec(memory_space=pltpu.VMEM),
        scratch_shapes=[
            pltpu.VMEM((N_DEV, m_per, n_per), jnp.bfloat16),
            pltpu.VMEM((N_DEV, m_per, n_per), jnp.bfloat16),
            pltpu.SemaphoreType.DMA((N_DEV,)),
            pltpu.SemaphoreType.DMA((N_DEV,)),
        ],
        compiler_params=pltpu.CompilerParams(
            vmem_limit_bytes=100 * 1024 * 1024,
        ),
    )(x, w_mat)


# device time: 35356 ns/iter; 1.2434x vs baseline; 1.2434x over previous
import jax
import jax.numpy as jnp
from jax import lax
from jax.experimental import pallas as pl
from jax.experimental.pallas import tpu as pltpu

N_DEV = 32
NT = 8
_GELU_C = 0.7978845608028654


def kernel(x, w_mat):
    m_per, k = x.shape
    _, n = w_mat.shape
    n_per = n // N_DEV
    n_t = n // NT
    d_per_t = N_DEV // NT

    def body(
        x_ref, w_ref, out_ref,
        xb_ref, ysend_ref, comm_ref, send_sems, recv_sems,
    ):
        t = pl.program_id(0)
        my = lax.axis_index("i")

        @pl.when(t == 0)
        def _():
            xb_ref[...] = x_ref[...].astype(jnp.bfloat16)

        wb = w_ref[...].astype(jnp.bfloat16)
        y = jnp.dot(xb_ref[...], wb, preferred_element_type=jnp.float32)
        y = 0.5 * y * (1.0 + jnp.tanh(_GELU_C * (y + 0.044715 * (y * y * y))))
        yb = y.astype(jnp.bfloat16)

        for jj in range(d_per_t):
            j = t * d_per_t + jj
            blk = yb[:, jj * n_per : (jj + 1) * n_per]

            @pl.when(j == my)
            def _():
                comm_ref[my] = blk

            @pl.when(j != my)
            def _():
                ysend_ref[j] = blk
                rdma = pltpu.make_async_remote_copy(
                    src_ref=ysend_ref.at[j],
                    dst_ref=comm_ref.at[my],
                    send_sem=send_sems.at[j],
                    recv_sem=recv_sems.at[my],
                    device_id=(j,),
                    device_id_type=pl.DeviceIdType.MESH,
                )
                rdma.start()

        @pl.when(t == NT - 1)
        def _():
            for s in range(N_DEV):
                @pl.when(s != my)
                def _():
                    recv = pltpu.make_async_remote_copy(
                        src_ref=comm_ref.at[s],
                        dst_ref=comm_ref.at[s],
                        send_sem=send_sems.at[s],
                        recv_sem=recv_sems.at[s],
                        device_id=(s,),
                        device_id_type=pl.DeviceIdType.MESH,
                    )
                    recv.wait_recv()

            out_ref[...] = comm_ref[...].reshape(N_DEV * m_per, n_per).astype(
                jnp.float32
            )

            for j in range(N_DEV):
                @pl.when(j != my)
                def _():
                    send = pltpu.make_async_remote_copy(
                        src_ref=ysend_ref.at[j],
                        dst_ref=comm_ref.at[my],
                        send_sem=send_sems.at[j],
                        recv_sem=recv_sems.at[my],
                        device_id=(j,),
                        device_id_type=pl.DeviceIdType.MESH,
                    )
                    send.wait_send()

    return pl.pallas_call(
        body,
        grid=(NT,),
        out_shape=jax.ShapeDtypeStruct((N_DEV * m_per, n_per), jnp.float32),
        in_specs=[
            pl.BlockSpec((m_per, k), lambda t: (0, 0)),
            pl.BlockSpec((k, n_t), lambda t: (0, t)),
        ],
        out_specs=pl.BlockSpec((N_DEV * m_per, n_per), lambda t: (0, 0)),
        scratch_shapes=[
            pltpu.VMEM((m_per, k), jnp.bfloat16),
            pltpu.VMEM((N_DEV, m_per, n_per), jnp.bfloat16),
            pltpu.VMEM((N_DEV, m_per, n_per), jnp.bfloat16),
            pltpu.SemaphoreType.DMA((N_DEV,)),
            pltpu.SemaphoreType.DMA((N_DEV,)),
        ],
        compiler_params=pltpu.CompilerParams(
            dimension_semantics=("arbitrary",),
            vmem_limit_bytes=100 * 1024 * 1024,
        ),
    )(x, w_mat)


# device time: 29984 ns/iter; 1.4661x vs baseline; 1.1792x over previous
import jax
import jax.numpy as jnp
from jax import lax
from jax.experimental import pallas as pl
from jax.experimental.pallas import tpu as pltpu

N_DEV = 32
NT = 4
_GELU_C = 0.7978845608028654


def kernel(x, w_mat):
    m_per, k = x.shape
    _, n = w_mat.shape
    n_per = n // N_DEV
    n_t = n // NT
    d_per_t = N_DEV // NT

    def body(
        x_ref, w_ref, out_ref,
        xb_ref, ysend_ref, comm_ref, send_sems, recv_sems,
    ):
        t = pl.program_id(0)
        my = lax.axis_index("i")
        bsem = pltpu.get_barrier_semaphore()

        @pl.when(t == 0)
        def _():
            def signal_body(d, carry):
                pl.semaphore_signal(
                    bsem, inc=1,
                    device_id=(lax.rem(my + d, N_DEV),),
                    device_id_type=pl.DeviceIdType.MESH,
                )
                return carry

            lax.fori_loop(1, N_DEV, signal_body, 0)
            xb_ref[...] = x_ref[...].astype(jnp.bfloat16)

        wb = w_ref[...].astype(jnp.bfloat16)
        y = jnp.dot(xb_ref[...], wb, preferred_element_type=jnp.float32)
        y = 0.5 * y * (1.0 + jnp.tanh(_GELU_C * (y + 0.044715 * (y * y * y))))
        yb = y.astype(jnp.bfloat16)
        h = m_per // 2
        for jj in range(d_per_t):
            j = t * d_per_t + jj
            blk = yb[:, jj * n_per : (jj + 1) * n_per]
            ysend_ref[j] = jnp.concatenate(
                [blk[0:h, :], blk[h : m_per, :]], axis=1
            )

        @pl.when(t == NT - 1)
        def _():
            pl.semaphore_wait(bsem, N_DEV - 1)

            def send_body(d, carry):
                j = lax.rem(my + d, N_DEV)
                rdma = pltpu.make_async_remote_copy(
                    src_ref=ysend_ref.at[j],
                    dst_ref=comm_ref.at[my],
                    send_sem=send_sems.at[j],
                    recv_sem=recv_sems.at[my],
                    device_id=(j,),
                    device_id_type=pl.DeviceIdType.MESH,
                )
                rdma.start()
                return carry

            lax.fori_loop(1, N_DEV, send_body, 0)

            own = ysend_ref[my]
            out_ref[pl.ds(my * m_per, h), :] = own[:, 0:h].astype(jnp.float32)
            out_ref[pl.ds(my * m_per + h, h), :] = own[:, h:m_per].astype(
                jnp.float32
            )

            def recv_body(d, carry):
                s = lax.rem(my + d, N_DEV)
                recv = pltpu.make_async_remote_copy(
                    src_ref=comm_ref.at[s],
                    dst_ref=comm_ref.at[s],
                    send_sem=send_sems.at[s],
                    recv_sem=recv_sems.at[s],
                    device_id=(s,),
                    device_id_type=pl.DeviceIdType.MESH,
                )
                recv.wait_recv()
                blk = comm_ref[s]
                out_ref[pl.ds(s * m_per, h), :] = blk[:, 0:h].astype(
                    jnp.float32
                )
                out_ref[pl.ds(s * m_per + h, h), :] = blk[:, h:m_per].astype(
                    jnp.float32
                )
                return carry

            lax.fori_loop(1, N_DEV, recv_body, 0)

            def drain_body(d, carry):
                j = lax.rem(my + d, N_DEV)
                send = pltpu.make_async_remote_copy(
                    src_ref=ysend_ref.at[j],
                    dst_ref=comm_ref.at[my],
                    send_sem=send_sems.at[j],
                    recv_sem=recv_sems.at[my],
                    device_id=(j,),
                    device_id_type=pl.DeviceIdType.MESH,
                )
                send.wait_send()
                return carry

            lax.fori_loop(1, N_DEV, drain_body, 0)

    return pl.pallas_call(
        body,
        grid=(NT,),
        out_shape=jax.ShapeDtypeStruct((N_DEV * m_per, n_per), jnp.float32),
        in_specs=[
            pl.BlockSpec((m_per, k), lambda t: (0, 0)),
            pl.BlockSpec((k, n_t), lambda t: (0, t)),
        ],
        out_specs=pl.BlockSpec((N_DEV * m_per, n_per), lambda t: (0, 0)),
        scratch_shapes=[
            pltpu.VMEM((m_per, k), jnp.bfloat16),
            pltpu.VMEM((N_DEV, n_per, m_per), jnp.bfloat16),
            pltpu.VMEM((N_DEV, n_per, m_per), jnp.bfloat16),
            pltpu.SemaphoreType.DMA((N_DEV,)),
            pltpu.SemaphoreType.DMA((N_DEV,)),
        ],
        compiler_params=pltpu.CompilerParams(
            dimension_semantics=("arbitrary",),
            vmem_limit_bytes=100 * 1024 * 1024,
            collective_id=0,
        ),
    )(x, w_mat)


# device time: 25525 ns/iter; 1.7223x vs baseline; 1.1747x over previous
import jax
import jax.numpy as jnp
from jax import lax
from jax.experimental import pallas as pl
from jax.experimental.pallas import tpu as pltpu

N_DEV = 32
NT = 4
K = 1
_GELU_C = 0.7978845608028654


def kernel(x, w_mat):
    m_per, k = x.shape
    _, n = w_mat.shape
    n_per = n // N_DEV
    n_t = n // NT
    d_per_t = N_DEV // NT
    h = m_per // 2

    def body(
        x_ref, w_ref, out_ref,
        xb_ref, ysend_ref, comm_ref, send_sems, recv_sems,
    ):
        t = pl.program_id(0)
        my = lax.axis_index("i")
        bsem = pltpu.get_barrier_semaphore()

        @pl.when(t == 0)
        def _():
            def signal_body(d, carry):
                pl.semaphore_signal(
                    bsem, inc=1,
                    device_id=(lax.rem(my + d, N_DEV),),
                    device_id_type=pl.DeviceIdType.MESH,
                )
                return carry

            lax.fori_loop(1, N_DEV, signal_body, 0)
            xb_ref[...] = x_ref[...].astype(jnp.bfloat16)

        wb = w_ref[...].astype(jnp.bfloat16)
        y = jnp.dot(xb_ref[...], wb, preferred_element_type=jnp.float32)
        y = 0.5 * y * (1.0 + jnp.tanh(_GELU_C * (y + 0.044715 * (y * y * y))))
        yb = y.astype(jnp.bfloat16)
        for jj in range(d_per_t):
            j = t * d_per_t + jj
            blk = yb[:, jj * n_per : (jj + 1) * n_per]
            ysend_ref[j] = jnp.concatenate(
                [blk[0:h, :], blk[h:m_per, :]], axis=1
            )

        def send_range(lo, hi):
            def send_body(d, carry):
                j = lax.rem(my + d, N_DEV)

                @pl.when(jnp.logical_and(j >= lo, j < hi))
                def _():
                    rdma = pltpu.make_async_remote_copy(
                        src_ref=ysend_ref.at[j],
                        dst_ref=comm_ref.at[my],
                        send_sem=send_sems.at[j],
                        recv_sem=recv_sems.at[my],
                        device_id=(j,),
                        device_id_type=pl.DeviceIdType.MESH,
                    )
                    rdma.start()
                return carry

            lax.fori_loop(1, N_DEV, send_body, 0)

        @pl.when(t == K)
        def _():
            pl.semaphore_wait(bsem, N_DEV - 1)
            send_range(0, (K + 1) * d_per_t)

        @pl.when(t > K)
        def _():
            send_range(t * d_per_t, (t + 1) * d_per_t)

        @pl.when(t == NT - 1)
        def _():
            own = ysend_ref[my]
            out_ref[pl.ds(my * m_per, h), :] = own[:, 0:h].astype(jnp.float32)
            out_ref[pl.ds(my * m_per + h, h), :] = own[:, h:m_per].astype(
                jnp.float32
            )

            def recv_body(d, carry):
                s = lax.rem(my + d, N_DEV)
                recv = pltpu.make_async_remote_copy(
                    src_ref=comm_ref.at[s],
                    dst_ref=comm_ref.at[s],
                    send_sem=send_sems.at[s],
                    recv_sem=recv_sems.at[s],
                    device_id=(s,),
                    device_id_type=pl.DeviceIdType.MESH,
                )
                recv.wait_recv()
                blk = comm_ref[s]
                out_ref[pl.ds(s * m_per, h), :] = blk[:, 0:h].astype(
                    jnp.float32
                )
                out_ref[pl.ds(s * m_per + h, h), :] = blk[:, h:m_per].astype(
                    jnp.float32
                )
                return carry

            lax.fori_loop(1, N_DEV, recv_body, 0)

            def drain_body(d, carry):
                j = lax.rem(my + d, N_DEV)
                send = pltpu.make_async_remote_copy(
                    src_ref=ysend_ref.at[j],
                    dst_ref=comm_ref.at[my],
                    send_sem=send_sems.at[j],
                    recv_sem=recv_sems.at[my],
                    device_id=(j,),
                    device_id_type=pl.DeviceIdType.MESH,
                )
                send.wait_send()
                return carry

            lax.fori_loop(1, N_DEV, drain_body, 0)

    return pl.pallas_call(
        body,
        grid=(NT,),
        out_shape=jax.ShapeDtypeStruct((N_DEV * m_per, n_per), jnp.float32),
        in_specs=[
            pl.BlockSpec((m_per, k), lambda t: (0, 0)),
            pl.BlockSpec((k, n_t), lambda t: (0, t)),
        ],
        out_specs=pl.BlockSpec((N_DEV * m_per, n_per), lambda t: (0, 0)),
        scratch_shapes=[
            pltpu.VMEM((m_per, k), jnp.bfloat16),
            pltpu.VMEM((N_DEV, n_per, m_per), jnp.bfloat16),
            pltpu.VMEM((N_DEV, n_per, m_per), jnp.bfloat16),
            pltpu.SemaphoreType.DMA((N_DEV,)),
            pltpu.SemaphoreType.DMA((N_DEV,)),
        ],
        compiler_params=pltpu.CompilerParams(
            dimension_semantics=("arbitrary",),
            vmem_limit_bytes=100 * 1024 * 1024,
            collective_id=0,
        ),
    )(x, w_mat)
